# baseline (speedup 1.0000x reference)
import jax
import jax.numpy as jnp
from jax import lax
from jax.experimental import pallas as pl
from jax.experimental.pallas import tpu as pltpu
from jax.experimental.pallas import tpu_sc as plsc

_info = plsc.get_sparse_core_info()
_NC, _NS, _L = _info.num_cores, _info.num_subcores, _info.num_lanes
_NW = _NC * _NS
_B = 16384
_V = 1000000
_D = 64
_BPW = _B // _NW
_CHUNK = 128
_DBLK = 8
_NG = _BPW // _L


def _glove_body(cw_hbm, tw_hbm, cooc_hbm, wt_hbm, dc_hbm, dt_hbm,
                embv_hbm, embu_hbm, vbt_hbm,
                out_hbm,
                cw_v, tw_v, cooc_v, wt_v, cb_v, tb_v, cembT, tembT,
                idxc, idxt, acc_v, sem):
    wid = lax.axis_index("s") * _NC + lax.axis_index("c")
    base = pl.multiple_of(wid * _BPW, _BPW)

    pltpu.sync_copy(cw_hbm.at[pl.ds(base, _BPW)], cw_v)
    pltpu.sync_copy(tw_hbm.at[pl.ds(base, _BPW)], tw_v)
    pltpu.sync_copy(cooc_hbm.at[pl.ds(base, _BPW)], cooc_v)
    pltpu.sync_copy(wt_hbm.at[pl.ds(base, _BPW)], wt_v)

    vb1 = vbt_hbm.at[0]
    ev1 = embv_hbm.at[0]
    eu1 = embu_hbm.at[0]

    bias_copies = []
    for c in range(_BPW // _CHUNK):
        s = pl.ds(c * _CHUNK, _CHUNK)
        bias_copies.append(pltpu.async_copy(vb1.at[cw_v.at[s]], cb_v.at[s], sem))
        bias_copies.append(pltpu.async_copy(vb1.at[tw_v.at[s]], tb_v.at[s], sem))

    for blk in range(_D // _DBLK):
        d0 = blk * _DBLK
        pltpu.sync_copy(dc_hbm.at[pl.ds(d0, _DBLK), pl.ds(base, _BPW)], idxc)
        pltpu.sync_copy(dt_hbm.at[pl.ds(d0, _DBLK), pl.ds(base, _BPW)], idxt)
        copies = []
        for dd in range(_DBLK):
            d = d0 + dd
            copies.append(pltpu.async_copy(
                ev1.at[idxc.at[dd]], cembT.at[d], sem))
            copies.append(pltpu.async_copy(
                eu1.at[idxt.at[dd]], tembT.at[d], sem))
        for cp in copies:
            cp.wait()
    for cp in bias_copies:
        cp.wait()

    acc = jnp.zeros((_L,), jnp.float32)

    def group(g, acc):
        e0 = pl.multiple_of(g * _L, _L)
        s = pl.ds(e0, _L)
        dot = cembT[0, s] * tembT[0, s]
        for d in range(1, _D):
            dot = dot + cembT[d, s] * tembT[d, s]
        err = dot + cb_v[s] + tb_v[s] - cooc_v[s]
        return acc + wt_v[s] * err * err

    acc = lax.fori_loop(0, _NG, group, acc)
    acc_v[...] = acc
    pltpu.sync_copy(acc_v, out_hbm.at[wid])


_glove_partials = pl.kernel(
    _glove_body,
    out_type=jax.ShapeDtypeStruct((_NW, _L), jnp.float32),
    mesh=plsc.VectorSubcoreMesh(core_axis_name="c", subcore_axis_name="s"),
    compiler_params=pltpu.CompilerParams(use_tc_tiling_on_sc=False),
    scratch_types=[
        pltpu.VMEM((_BPW,), jnp.int32),
        pltpu.VMEM((_BPW,), jnp.int32),
        pltpu.VMEM((_BPW,), jnp.float32),
        pltpu.VMEM((_BPW,), jnp.float32),
        pltpu.VMEM((_BPW,), jnp.float32),
        pltpu.VMEM((_BPW,), jnp.float32),
        pltpu.VMEM((_D, _BPW), jnp.float32),
        pltpu.VMEM((_D, _BPW), jnp.float32),
        pltpu.VMEM((_DBLK, _BPW), jnp.int32),
        pltpu.VMEM((_DBLK, _BPW), jnp.int32),
        pltpu.VMEM((_L,), jnp.float32),
        pltpu.SemaphoreType.DMA,
    ],
)


def _sum_body(x_ref, o_ref):
    o_ref[...] = jnp.sum(x_ref[...], keepdims=True)


def kernel(center_words, target_words, coocs, weights, emb_v, emb_u, v_bias,
           u_bias):
    del u_bias
    cw = center_words.reshape(_B)
    tw = target_words.reshape(_B)
    cooc = coocs.reshape(_B)
    wt = weights.reshape(_B)
    doff = jnp.arange(_D, dtype=jnp.int32)[:, None] * _V
    partials = _glove_partials(cw, tw, cooc, wt,
                               cw[None, :] + doff, tw[None, :] + doff,
                               emb_v.T.reshape(1, _V * _D),
                               emb_u.T.reshape(1, _V * _D),
                               v_bias.T)
    total = pl.pallas_call(
        _sum_body,
        out_shape=jax.ShapeDtypeStruct((1, 1), jnp.float32),
    )(partials)
    return total[0, 0]


# speedup vs baseline: 9.4728x; 9.4728x over previous
import jax
import jax.numpy as jnp
from jax import lax
from jax.experimental import pallas as pl
from jax.experimental.pallas import tpu as pltpu
from jax.experimental.pallas import tpu_sc as plsc

_info = plsc.get_sparse_core_info()
_NC, _NS, _L = _info.num_cores, _info.num_subcores, _info.num_lanes
_NW = _NC * _NS
_B = 16384
_V = 1000000
_D = 64
_BPW = _B // _NW
_CHUNK = 128
_NH = 2
_HPW = _BPW // _NH
_NG = _HPW // _L
_BITREV = [0, 8, 4, 12, 2, 10, 6, 14, 1, 9, 5, 13, 3, 11, 7, 15]


def _glove_body(cw_hbm, tw_hbm, cooc_hbm, wt_hbm, embv_hbm, embu_hbm,
                vbt_hbm,
                out_hbm,
                cw_v, tw_v, cooc_v, wt_v, cb_v, tb_v, cemb, temb,
                rbuf, acc_v, sem):
    wid = lax.axis_index("s") * _NC + lax.axis_index("c")
    base = pl.multiple_of(wid * _BPW, _BPW)

    pltpu.sync_copy(cw_hbm.at[pl.ds(base, _BPW)], cw_v)
    pltpu.sync_copy(tw_hbm.at[pl.ds(base, _BPW)], tw_v)
    pltpu.sync_copy(cooc_hbm.at[pl.ds(base, _BPW)], cooc_v)
    pltpu.sync_copy(wt_hbm.at[pl.ds(base, _BPW)], wt_v)

    vb1 = vbt_hbm.at[0]
    bias_copies = []
    for c in range(_BPW // _CHUNK):
        s = pl.ds(c * _CHUNK, _CHUNK)
        bias_copies.append(pltpu.async_copy(vb1.at[cw_v.at[s]], cb_v.at[s], sem))
        bias_copies.append(pltpu.async_copy(vb1.at[tw_v.at[s]], tb_v.at[s], sem))

    lane = lax.iota(jnp.int32, _L)
    masks = {h: (lane & h) == 0 for h in (8, 4, 2, 1)}

    acc = jnp.zeros((_L,), jnp.float32)
    for half in range(_NH):
        emb_copies = []
        for c in range(_HPW // _CHUNK):
            si = pl.ds(half * _HPW + c * _CHUNK, _CHUNK)
            so = pl.ds(c * _CHUNK, _CHUNK)
            emb_copies.append(pltpu.async_copy(embv_hbm.at[cw_v.at[si]], cemb.at[so], sem))
            emb_copies.append(pltpu.async_copy(embu_hbm.at[tw_v.at[si]], temb.at[so], sem))
        for cp in emb_copies:
            cp.wait()
        if half == 0:
            for cp in bias_copies:
                cp.wait()

        def group(g, acc):
            b0 = pl.multiple_of(g * _L, _L)
            vals = []
            for i, j in enumerate(_BITREV):
                b = b0 + j
                p = cemb[b, pl.ds(0, _L)] * temb[b, pl.ds(0, _L)]
                for k in range(1, _D // _L):
                    p = p + cemb[b, pl.ds(k * _L, _L)] * temb[b, pl.ds(k * _L, _L)]
                sb = 8 + 32 * i
                rbuf[pl.ds(sb, _L)] = p
                vals.append((p, sb))
            slot = _L
            for h in (8, 4, 2, 1):
                m = masks[h]
                nxt = []
                for t in range(len(vals) // 2):
                    (av, ab), (bv, bb) = vals[2 * t], vals[2 * t + 1]
                    a_rot = rbuf[pl.ds(ab + h, _L)]
                    b_rot = rbuf[pl.ds(bb - h, _L)]
                    cv = jnp.where(m, av + a_rot, bv + b_rot)
                    sb2 = -1
                    if h > 1:
                        sb2 = 8 + 32 * slot
                        slot += 1
                        rbuf[pl.ds(sb2, _L)] = cv
                    nxt.append((cv, sb2))
                vals = nxt
            dotv = vals[0][0]
            e0 = pl.multiple_of(half * _HPW + g * _L, _L)
            cb = cb_v[pl.ds(e0, _L)]
            tb = tb_v[pl.ds(e0, _L)]
            cooc = cooc_v[pl.ds(e0, _L)]
            wt = wt_v[pl.ds(e0, _L)]
            err = dotv + cb + tb - cooc
            return acc + wt * err * err

        acc = lax.fori_loop(0, _NG, group, acc)

    acc_v[...] = acc
    pltpu.sync_copy(acc_v, out_hbm.at[wid])


_glove_partials = pl.kernel(
    _glove_body,
    out_type=jax.ShapeDtypeStruct((_NW, _L), jnp.float32),
    mesh=plsc.VectorSubcoreMesh(core_axis_name="c", subcore_axis_name="s"),
    compiler_params=pltpu.CompilerParams(use_tc_tiling_on_sc=False),
    scratch_types=[
        pltpu.VMEM((_BPW,), jnp.int32),
        pltpu.VMEM((_BPW,), jnp.int32),
        pltpu.VMEM((_BPW,), jnp.float32),
        pltpu.VMEM((_BPW,), jnp.float32),
        pltpu.VMEM((_BPW,), jnp.float32),
        pltpu.VMEM((_BPW,), jnp.float32),
        pltpu.VMEM((_HPW, _D), jnp.float32),
        pltpu.VMEM((_HPW, _D), jnp.float32),
        pltpu.VMEM((1024,), jnp.float32),
        pltpu.VMEM((_L,), jnp.float32),
        pltpu.SemaphoreType.DMA,
    ],
)


def _sum_body(x_ref, o_ref):
    o_ref[...] = jnp.sum(x_ref[...], keepdims=True)


def kernel(center_words, target_words, coocs, weights, emb_v, emb_u, v_bias,
           u_bias):
    del u_bias
    cw = center_words.reshape(_B)
    tw = target_words.reshape(_B)
    cooc = coocs.reshape(_B)
    wt = weights.reshape(_B)
    partials = _glove_partials(cw, tw, cooc, wt, emb_v, emb_u, v_bias.T)
    total = pl.pallas_call(
        _sum_body,
        out_shape=jax.ShapeDtypeStruct((1, 1), jnp.float32),
    )(partials)
    return total[0, 0]
